# Initial kernel scaffold; baseline (speedup 1.0000x reference)
import functools

import jax
import jax.numpy as jnp
from jax import lax
from jax.experimental import pallas as pl
from jax.experimental.pallas import tpu as pltpu
from jax.experimental.pallas import tpu_sc as plsc

N_NODES = 10000
N_EDGES = 320000
D_PAD = 16
IDX_W = 128
N_ROWS = N_EDGES // IDX_W
NC, NS = 2, 16
NW = NC * NS
NODES_PER_TILE = N_NODES // NS


def _matmul_body(x_ref, w_ref, o_ref):
    o_ref[...] = jnp.dot(x_ref[...], w_ref[...],
                         preferred_element_type=jnp.float32)


def _tc_matmul(x, w):
    return pl.pallas_call(
        _matmul_body,
        out_shape=jax.ShapeDtypeStruct((N_NODES, D_PAD), jnp.float32),
    )(x, w)


def _sc_body(g_hbm, src_hbm, dst_hbm, ones_hbm, zero_hbm, out_hbm,
             acc_sh, src_v, dst_v, rows_v, ones_v, sem):
    c = lax.axis_index("c")
    s = lax.axis_index("s")
    wid = s * NC + c

    pltpu.sync_copy(zero_hbm.at[pl.ds(s * NODES_PER_TILE, NODES_PER_TILE)],
                    acc_sh.at[pl.ds(s * NODES_PER_TILE, NODES_PER_TILE)])
    pltpu.sync_copy(ones_hbm, ones_v)
    plsc.subcore_barrier()

    r0 = (N_ROWS * wid) // NW
    r1 = (N_ROWS * (wid + 1)) // NW

    def body(r, carry):
        pltpu.sync_copy(src_hbm.at[r], src_v)
        pltpu.sync_copy(dst_hbm.at[r], dst_v)
        pltpu.async_copy(g_hbm.at[src_v], rows_v, sem).wait()
        pltpu.sync_copy(rows_v, acc_sh.at[dst_v], add=True)
        pltpu.sync_copy(ones_v, acc_sh.at[src_v], add=True)
        return carry

    lax.fori_loop(r0, r1, body, 0)
    plsc.subcore_barrier()

    pltpu.sync_copy(acc_sh.at[pl.ds(s * NODES_PER_TILE, NODES_PER_TILE)],
                    out_hbm.at[c, pl.ds(s * NODES_PER_TILE, NODES_PER_TILE)])


_sc_scatter = functools.partial(
    pl.kernel,
    out_type=jax.ShapeDtypeStruct((NC, N_NODES, D_PAD), jnp.float32),
    mesh=plsc.VectorSubcoreMesh(core_axis_name="c", subcore_axis_name="s"),
    scratch_types=[
        pltpu.VMEM_SHARED((N_NODES, D_PAD), jnp.float32),
        pltpu.VMEM((IDX_W,), jnp.int32),
        pltpu.VMEM((IDX_W,), jnp.int32),
        pltpu.VMEM((IDX_W, D_PAD), jnp.float32),
        pltpu.VMEM((IDX_W, D_PAD), jnp.float32),
        pltpu.SemaphoreType.DMA,
    ],
)(_sc_body)


def _finalize_body(slabs_ref, w2_ref, wfc_ref, o_ref):
    t = jnp.maximum(slabs_ref[0] + slabs_ref[1], 0.0)
    s = jnp.sum(t * t[:, 10:11], axis=0, keepdims=True)
    z = jnp.dot(jnp.dot(s * (1.0 / N_NODES), w2_ref[...],
                        preferred_element_type=jnp.float32),
                wfc_ref[...], preferred_element_type=jnp.float32)
    o_ref[...] = 1.0 / (1.0 + jnp.exp(-z))


def _tc_finalize(slabs, w2p, wfcp):
    return pl.pallas_call(
        _finalize_body,
        out_shape=jax.ShapeDtypeStruct((1, 1), jnp.float32),
    )(slabs, w2p, wfcp)


def kernel(features, edge_index, W1, W2, Wfc):
    src = edge_index[0].reshape(N_ROWS, IDX_W)
    dst = edge_index[1].reshape(N_ROWS, IDX_W)
    w1p = jnp.pad(W1, ((0, 0), (0, D_PAD - W1.shape[1])))
    w2p = jnp.zeros((D_PAD, D_PAD), jnp.float32).at[:10, :10].set(W2)
    wfcp = jnp.zeros((D_PAD, 1), jnp.float32).at[:10].set(Wfc)
    ones_rows = jnp.zeros((IDX_W, D_PAD), jnp.float32).at[:, 10].set(1.0)
    zero_init = jnp.zeros((N_NODES, D_PAD), jnp.float32)

    g = _tc_matmul(features, w1p)
    slabs = _sc_scatter(g, src, dst, ones_rows, zero_init)
    return _tc_finalize(slabs, w2p, wfcp)


# speedup vs baseline: 10.5566x; 10.5566x over previous
import functools

import jax
import jax.numpy as jnp
from jax import lax
from jax.experimental import pallas as pl
from jax.experimental.pallas import tpu as pltpu
from jax.experimental.pallas import tpu_sc as plsc

N_NODES = 10000
N_EDGES = 320000
N_PAD = 10240
D_PAD = 128
IDX_W = 128
NC, NS = 2, 16
NW = NC * NS
N_ROWS = 2560
E_PAD = N_ROWS * IDX_W
RT = N_ROWS // NW
CH = 40
NODES_PER_TILE = N_PAD // NS
F_TILE = 1280
F_GRID = N_PAD // F_TILE


def _matmul_body(x_ref, w_ref, o_ref):
    o_ref[...] = jnp.dot(x_ref[...], w_ref[...],
                         preferred_element_type=jnp.float32)


def _tc_matmul(x, w):
    return pl.pallas_call(
        _matmul_body,
        out_shape=jax.ShapeDtypeStruct((N_PAD, D_PAD), jnp.float32),
    )(x, w)


def _sc_body(g_hbm, src_hbm, dst_hbm, ones_hbm, zero_hbm, out_hbm,
             acc_sh, idx_src, idx_dst, rows_v, ones_v, sem):
    c = lax.axis_index("c")
    s = lax.axis_index("s")
    wid = s * NC + c

    pltpu.sync_copy(zero_hbm.at[pl.ds(s * NODES_PER_TILE, NODES_PER_TILE)],
                    acc_sh.at[pl.ds(s * NODES_PER_TILE, NODES_PER_TILE)])
    pltpu.sync_copy(ones_hbm, ones_v)
    plsc.subcore_barrier()

    def body(j, carry):
        pltpu.async_copy(g_hbm.at[idx_src.at[j]], rows_v, sem).wait()
        pltpu.sync_copy(rows_v, acc_sh.at[idx_dst.at[j]], add=True)
        pltpu.sync_copy(ones_v, acc_sh.at[idx_src.at[j]], add=True)
        return carry

    for h in range(RT // CH):
        pltpu.sync_copy(src_hbm.at[pl.ds(wid * RT + h * CH, CH)], idx_src)
        pltpu.sync_copy(dst_hbm.at[pl.ds(wid * RT + h * CH, CH)], idx_dst)
        lax.fori_loop(0, CH, body, 0)
    plsc.subcore_barrier()

    pltpu.sync_copy(acc_sh.at[pl.ds(s * NODES_PER_TILE, NODES_PER_TILE)],
                    out_hbm.at[c, pl.ds(s * NODES_PER_TILE, NODES_PER_TILE)])


_sc_scatter = functools.partial(
    pl.kernel,
    out_type=jax.ShapeDtypeStruct((NC, N_PAD, D_PAD), jnp.float32),
    mesh=plsc.VectorSubcoreMesh(core_axis_name="c", subcore_axis_name="s"),
    scratch_types=[
        pltpu.VMEM_SHARED((N_PAD, D_PAD), jnp.float32),
        pltpu.VMEM((CH, IDX_W), jnp.int32),
        pltpu.VMEM((CH, IDX_W), jnp.int32),
        pltpu.VMEM((IDX_W, D_PAD), jnp.float32),
        pltpu.VMEM((IDX_W, D_PAD), jnp.float32),
        pltpu.SemaphoreType.DMA,
    ],
)(_sc_body)


def _finalize_body(slabs_ref, w2_ref, wfc_ref, o_ref, s_acc):
    i = pl.program_id(0)

    @pl.when(i == 0)
    def _init():
        s_acc[...] = jnp.zeros_like(s_acc)

    t = jnp.maximum(slabs_ref[0] + slabs_ref[1], 0.0)
    s_acc[...] += jnp.sum(t * t[:, 10:11], axis=0, keepdims=True)

    @pl.when(i == F_GRID - 1)
    def _done():
        z = jnp.dot(jnp.dot(s_acc[...] * (1.0 / N_NODES), w2_ref[...],
                            preferred_element_type=jnp.float32),
                    wfc_ref[...], preferred_element_type=jnp.float32)
        o_ref[...] = 1.0 / (1.0 + jnp.exp(-z))


def _tc_finalize(slabs, w2p, wfcp):
    return pl.pallas_call(
        _finalize_body,
        grid=(F_GRID,),
        in_specs=[
            pl.BlockSpec((NC, F_TILE, D_PAD), lambda i: (0, i, 0)),
            pl.BlockSpec((D_PAD, D_PAD), lambda i: (0, 0)),
            pl.BlockSpec((D_PAD, 1), lambda i: (0, 0)),
        ],
        out_specs=pl.BlockSpec((1, 1), lambda i: (0, 0)),
        out_shape=jax.ShapeDtypeStruct((1, 1), jnp.float32),
        scratch_shapes=[pltpu.VMEM((1, D_PAD), jnp.float32)],
    )(slabs, w2p, wfcp)


def kernel(features, edge_index, W1, W2, Wfc):
    pad_e = E_PAD - N_EDGES
    pad_idx = N_NODES + (jnp.arange(pad_e, dtype=jnp.int32)
                         % (N_PAD - N_NODES))
    src = jnp.concatenate([edge_index[0], pad_idx]).reshape(N_ROWS, IDX_W)
    dst = jnp.concatenate([edge_index[1], pad_idx]).reshape(N_ROWS, IDX_W)
    xp = jnp.pad(features, ((0, N_PAD - N_NODES), (0, 0)))
    w1p = jnp.pad(W1, ((0, 0), (0, D_PAD - W1.shape[1])))
    w2p = jnp.zeros((D_PAD, D_PAD), jnp.float32).at[:10, :10].set(W2)
    wfcp = jnp.zeros((D_PAD, 1), jnp.float32).at[:10].set(Wfc)
    ones_rows = jnp.zeros((IDX_W, D_PAD), jnp.float32).at[:, 10].set(1.0)
    zero_init = jnp.zeros((N_PAD, D_PAD), jnp.float32)

    g = _tc_matmul(xp, w1p)
    slabs = _sc_scatter(g, src, dst, ones_rows, zero_init)
    return _tc_finalize(slabs, w2p, wfcp)


# speedup vs baseline: 12.5872x; 1.1924x over previous
import functools

import jax
import jax.numpy as jnp
from jax import lax
from jax.experimental import pallas as pl
from jax.experimental.pallas import tpu as pltpu
from jax.experimental.pallas import tpu_sc as plsc

N_NODES = 10000
N_EDGES = 320000
N_PAD = 10240
D_PAD = 128
IDX_W = 128
NC, NS = 2, 16
NW = NC * NS
N_ROWS = 2560
E_PAD = N_ROWS * IDX_W
RT = N_ROWS // NW
CH = 40
NODES_PER_TILE = N_PAD // NS
F_TILE = 2048
F_GRID = N_PAD // F_TILE


def _matmul_body(x_ref, w_ref, o_ref):
    o_ref[...] = jnp.dot(x_ref[...], w_ref[...],
                         preferred_element_type=jnp.float32)


def _tc_matmul(x, w):
    return pl.pallas_call(
        _matmul_body,
        out_shape=jax.ShapeDtypeStruct((N_PAD, D_PAD), jnp.float32),
    )(x, w)


def _sc_body(g_hbm, src_hbm, dst_hbm, ones_hbm, zero_hbm, out_hbm,
             acc_sh, idx_src, idx_dst, rows_v, ones_v, sem):
    c = lax.axis_index("c")
    s = lax.axis_index("s")
    wid = s * NC + c

    pltpu.sync_copy(zero_hbm.at[pl.ds(s * NODES_PER_TILE, NODES_PER_TILE)],
                    acc_sh.at[pl.ds(s * NODES_PER_TILE, NODES_PER_TILE)])
    pltpu.sync_copy(ones_hbm, ones_v)
    plsc.subcore_barrier()

    def body(j, carry):
        cp = pltpu.async_copy(g_hbm.at[idx_src.at[j]], rows_v, sem)
        pltpu.sync_copy(ones_v, acc_sh.at[idx_src.at[j]], add=True)
        cp.wait()
        pltpu.sync_copy(rows_v, acc_sh.at[idx_dst.at[j]], add=True)
        return carry

    for h in range(RT // CH):
        pltpu.sync_copy(src_hbm.at[pl.ds(wid * RT + h * CH, CH)], idx_src)
        pltpu.sync_copy(dst_hbm.at[pl.ds(wid * RT + h * CH, CH)], idx_dst)
        lax.fori_loop(0, CH, body, 0)
    plsc.subcore_barrier()

    pltpu.sync_copy(acc_sh.at[pl.ds(s * NODES_PER_TILE, NODES_PER_TILE)],
                    out_hbm.at[c, pl.ds(s * NODES_PER_TILE, NODES_PER_TILE)])


_sc_scatter = functools.partial(
    pl.kernel,
    out_type=jax.ShapeDtypeStruct((NC, N_PAD, D_PAD), jnp.float32),
    mesh=plsc.VectorSubcoreMesh(core_axis_name="c", subcore_axis_name="s"),
    scratch_types=[
        pltpu.VMEM_SHARED((N_PAD, D_PAD), jnp.float32),
        pltpu.VMEM((CH, IDX_W), jnp.int32),
        pltpu.VMEM((CH, IDX_W), jnp.int32),
        pltpu.VMEM((IDX_W, D_PAD), jnp.float32),
        pltpu.VMEM((IDX_W, D_PAD), jnp.float32),
        pltpu.SemaphoreType.DMA,
    ],
)(_sc_body)


def _finalize_body(slabs_ref, w2_ref, wfc_ref, o_ref, s_acc):
    i = pl.program_id(0)

    @pl.when(i == 0)
    def _init():
        s_acc[...] = jnp.zeros_like(s_acc)

    t = jnp.maximum(slabs_ref[0] + slabs_ref[1], 0.0)
    s_acc[...] += jnp.sum(t * t[:, 10:11], axis=0, keepdims=True)

    @pl.when(i == F_GRID - 1)
    def _done():
        z = jnp.dot(jnp.dot(s_acc[...] * (1.0 / N_NODES), w2_ref[...],
                            preferred_element_type=jnp.float32),
                    wfc_ref[...], preferred_element_type=jnp.float32)
        o_ref[...] = 1.0 / (1.0 + jnp.exp(-z))


def _tc_finalize(slabs, w2p, wfcp):
    return pl.pallas_call(
        _finalize_body,
        grid=(F_GRID,),
        in_specs=[
            pl.BlockSpec((NC, F_TILE, D_PAD), lambda i: (0, i, 0)),
            pl.BlockSpec((D_PAD, D_PAD), lambda i: (0, 0)),
            pl.BlockSpec((D_PAD, 1), lambda i: (0, 0)),
        ],
        out_specs=pl.BlockSpec((1, 1), lambda i: (0, 0)),
        out_shape=jax.ShapeDtypeStruct((1, 1), jnp.float32),
        scratch_shapes=[pltpu.VMEM((1, D_PAD), jnp.float32)],
    )(slabs, w2p, wfcp)


def kernel(features, edge_index, W1, W2, Wfc):
    pad_e = E_PAD - N_EDGES
    pad_idx = N_NODES + (jnp.arange(pad_e, dtype=jnp.int32)
                         % (N_PAD - N_NODES))
    src = jnp.concatenate([edge_index[0], pad_idx]).reshape(N_ROWS, IDX_W)
    dst = jnp.concatenate([edge_index[1], pad_idx]).reshape(N_ROWS, IDX_W)
    xp = jnp.pad(features, ((0, N_PAD - N_NODES), (0, 0)))
    w1p = jnp.pad(W1, ((0, 0), (0, D_PAD - W1.shape[1])))
    w2p = jnp.zeros((D_PAD, D_PAD), jnp.float32).at[:10, :10].set(W2)
    wfcp = jnp.zeros((D_PAD, 1), jnp.float32).at[:10].set(Wfc)
    ones_rows = jnp.zeros((IDX_W, D_PAD), jnp.float32).at[:, 10].set(1.0)
    zero_init = jnp.zeros((N_PAD, D_PAD), jnp.float32)

    g = _tc_matmul(xp, w1p)
    slabs = _sc_scatter(g, src, dst, ones_rows, zero_init)
    return _tc_finalize(slabs, w2p, wfcp)
